# Initial kernel scaffold; baseline (speedup 1.0000x reference)
import jax
import jax.numpy as jnp
from jax.experimental import pallas as pl


def kernel(preds, attrs):
    raise NotImplementedError("write your pallas kernel here")


# speedup vs baseline: 22.5563x; 22.5563x over previous
import functools

import jax
import jax.numpy as jnp
from jax import lax
from jax.experimental import pallas as pl
from jax.experimental.pallas import tpu as pltpu
from jax.experimental.pallas import tpu_sc as plsc

_NC = 2
_NS = 16
_L = 16
_NW = _NC * _NS
_UNROLL = 4


@functools.lru_cache(maxsize=None)
def _make_hist_kernel(n: int):
    chunk = n // _NW
    n_vec = chunk // _L
    mesh = plsc.VectorSubcoreMesh(
        core_axis_name="c", subcore_axis_name="s", num_cores=_NC,
        num_subcores=_NS)

    @functools.partial(
        pl.kernel,
        out_type=jax.ShapeDtypeStruct((2 * _NW * _L,), jnp.float32),
        mesh=mesh,
        compiler_params=pltpu.CompilerParams(needs_layout_passes=False),
        scratch_types=[
            pltpu.VMEM((chunk,), jnp.int32),
            pltpu.VMEM((chunk,), jnp.int32),
            pltpu.VMEM((2 * _L * _L,), jnp.float32),
            pltpu.VMEM((_L,), jnp.float32),
        ],
    )
    def hist_kernel(preds_hbm, attrs_hbm, out_hbm, preds_v, attrs_v, hist_v,
                    row_v):
        c = lax.axis_index("c")
        s = lax.axis_index("s")
        wid = s * _NC + c
        base = wid * chunk

        pltpu.sync_copy(preds_hbm.at[pl.ds(base, chunk)], preds_v)
        pltpu.sync_copy(attrs_hbm.at[pl.ds(base, chunk)], attrs_v)

        zeros = jnp.zeros((_L,), jnp.float32)
        for i in range(2 * _L):
            hist_v[pl.ds(i * _L, _L)] = zeros

        lane16 = lax.iota(jnp.int32, _L) * _L
        ones = jnp.ones((_L,), jnp.float32)

        def body(i, carry):
            off = i * (_L * _UNROLL)
            for u in range(_UNROLL):
                p = preds_v[pl.ds(off + u * _L, _L)]
                a = attrs_v[pl.ds(off + u * _L, _L)]
                addr = a * 256 + (lane16 + p)
                plsc.addupdate_scatter(hist_v, [addr], ones)
            return carry

        lax.fori_loop(0, n_vec // _UNROLL, body, 0)

        for a in range(2):
            acc = hist_v[pl.ds(a * 256, _L)]
            for l in range(1, _L):
                acc = acc + hist_v[pl.ds(a * 256 + l * _L, _L)]
            row_v[...] = acc
            pltpu.sync_copy(row_v,
                            out_hbm.at[pl.ds((a * _NW + wid) * _L, _L)])

    return hist_kernel


def _spd_body(x_ref, o_ref):
    x = x_ref[...]
    h0 = jnp.sum(x[0:_NW, :], axis=0, keepdims=True)
    h1 = jnp.sum(x[_NW:, :], axis=0, keepdims=True)
    n0 = jnp.sum(h0)
    n1 = jnp.sum(h1)
    d = h0 / n0 - h1 / n1
    o_ref[0, 0] = jnp.sum(d * d)


@jax.jit
def kernel(preds, attrs):
    n = preds.shape[0]
    partial = _make_hist_kernel(n)(preds, attrs)
    partial = partial.reshape(2 * _NW, _L)
    spd = pl.pallas_call(
        _spd_body,
        out_shape=jax.ShapeDtypeStruct((1, 1), jnp.float32),
        out_specs=pl.BlockSpec(memory_space=pltpu.SMEM),
    )(partial)
    return spd[0, 0]


# speedup vs baseline: 38.0153x; 1.6854x over previous
import functools

import jax
import jax.numpy as jnp
from jax import lax
from jax.experimental import pallas as pl
from jax.experimental.pallas import tpu as pltpu
from jax.experimental.pallas import tpu_sc as plsc

_NC = 2
_NS = 16
_L = 16
_NW = _NC * _NS
_SUB = 4
_UNROLL = 8


@functools.lru_cache(maxsize=None)
def _make_hist_kernel(n: int):
    chunk = n // _NW
    sub = chunk // _SUB
    mesh = plsc.VectorSubcoreMesh(
        core_axis_name="c", subcore_axis_name="s", num_cores=_NC,
        num_subcores=_NS)

    @functools.partial(
        pl.kernel,
        out_type=jax.ShapeDtypeStruct((2 * _NW * _L,), jnp.float32),
        mesh=mesh,
        compiler_params=pltpu.CompilerParams(needs_layout_passes=False),
        scratch_types=[
            pltpu.VMEM((chunk,), jnp.int32),
            pltpu.VMEM((chunk,), jnp.int32),
            pltpu.VMEM((2 * _L * _L,), jnp.float32),
            pltpu.VMEM((_L,), jnp.float32),
            pltpu.SemaphoreType.DMA((_SUB,)),
        ],
    )
    def hist_kernel(preds_hbm, attrs_hbm, out_hbm, preds_v, attrs_v, hist_v,
                    row_v, sems):
        c = lax.axis_index("c")
        s = lax.axis_index("s")
        wid = s * _NC + c
        base = wid * chunk

        descs = []
        for g in range(_SUB):
            src_p = preds_hbm.at[pl.ds(base + g * sub, sub)]
            src_a = attrs_hbm.at[pl.ds(base + g * sub, sub)]
            descs.append(
                (pltpu.async_copy(src_p, preds_v.at[pl.ds(g * sub, sub)],
                                  sems.at[g]),
                 pltpu.async_copy(src_a, attrs_v.at[pl.ds(g * sub, sub)],
                                  sems.at[g])))

        zeros = jnp.zeros((_L,), jnp.float32)
        for i in range(2 * _L):
            hist_v[pl.ds(i * _L, _L)] = zeros

        lane16 = lax.iota(jnp.int32, _L) * _L
        ones = jnp.ones((_L,), jnp.float32)

        for g in range(_SUB):
            descs[g][0].wait()
            descs[g][1].wait()

            @functools.partial(
                plsc.parallel_loop, 0, sub // _L, unroll=_UNROLL)
            def body(i, g=g):
                off = g * sub + i * _L
                p = preds_v[pl.ds(off, _L)]
                a = attrs_v[pl.ds(off, _L)]
                addr = a * 256 + (lane16 + p)
                plsc.addupdate_scatter(hist_v, [addr], ones)

        for a in range(2):
            acc = hist_v[pl.ds(a * 256, _L)]
            for l in range(1, _L):
                acc = acc + hist_v[pl.ds(a * 256 + l * _L, _L)]
            row_v[...] = acc
            pltpu.sync_copy(row_v,
                            out_hbm.at[pl.ds((a * _NW + wid) * _L, _L)])

    return hist_kernel


def _spd_body(x_ref, o_ref):
    x = x_ref[...]
    h0 = jnp.sum(x[0:_NW, :], axis=0, keepdims=True)
    h1 = jnp.sum(x[_NW:, :], axis=0, keepdims=True)
    n0 = jnp.sum(h0)
    n1 = jnp.sum(h1)
    d = h0 / n0 - h1 / n1
    o_ref[0, 0] = jnp.sum(d * d)


@jax.jit
def kernel(preds, attrs):
    n = preds.shape[0]
    partial = _make_hist_kernel(n)(preds, attrs)
    partial = partial.reshape(2 * _NW, _L)
    spd = pl.pallas_call(
        _spd_body,
        out_shape=jax.ShapeDtypeStruct((1, 1), jnp.float32),
        out_specs=pl.BlockSpec(memory_space=pltpu.SMEM),
    )(partial)
    return spd[0, 0]


# speedup vs baseline: 40.1213x; 1.0554x over previous
import functools

import jax
import jax.numpy as jnp
from jax import lax
from jax.experimental import pallas as pl
from jax.experimental.pallas import tpu as pltpu
from jax.experimental.pallas import tpu_sc as plsc

_NC = 2
_NS = 16
_L = 16
_NW = _NC * _NS
_SUB = 4
_UNROLL = 8


@functools.lru_cache(maxsize=None)
def _make_hist_kernel(n: int):
    chunk = n // _NW
    sub = chunk // _SUB
    mesh = plsc.VectorSubcoreMesh(
        core_axis_name="c", subcore_axis_name="s", num_cores=_NC,
        num_subcores=_NS)

    @functools.partial(
        pl.kernel,
        out_type=jax.ShapeDtypeStruct((2 * _NW * _L,), jnp.float32),
        mesh=mesh,
        compiler_params=pltpu.CompilerParams(needs_layout_passes=False),
        scratch_types=[
            pltpu.VMEM((chunk,), jnp.int32),
            pltpu.VMEM((chunk,), jnp.int32),
            pltpu.VMEM((2 * _L * _L,), jnp.float32),
            pltpu.VMEM((_L,), jnp.float32),
            pltpu.SemaphoreType.DMA((_SUB,)),
        ],
    )
    def hist_kernel(preds_hbm, attrs_hbm, out_hbm, preds_v, attrs_v, hist_v,
                    row_v, sems):
        c = lax.axis_index("c")
        s = lax.axis_index("s")
        wid = s * _NC + c
        base = wid * chunk

        descs = []
        for g in range(_SUB):
            src_p = preds_hbm.at[pl.ds(base + g * sub, sub)]
            src_a = attrs_hbm.at[pl.ds(base + g * sub, sub)]
            descs.append(
                (pltpu.async_copy(src_p, preds_v.at[pl.ds(g * sub, sub)],
                                  sems.at[g]),
                 pltpu.async_copy(src_a, attrs_v.at[pl.ds(g * sub, sub)],
                                  sems.at[g])))

        zeros = jnp.zeros((_L,), jnp.float32)
        for i in range(2 * _L):
            hist_v[pl.ds(i * _L, _L)] = zeros

        lane16 = lax.iota(jnp.int32, _L) * _L
        ones = jnp.ones((_L,), jnp.float32)

        for g in range(_SUB):
            descs[g][0].wait()
            descs[g][1].wait()

            @functools.partial(
                plsc.parallel_loop, 0, sub // _L, unroll=_UNROLL)
            def body(i, g=g):
                off = g * sub + i * _L
                p = preds_v[pl.ds(off, _L)]
                a = attrs_v[pl.ds(off, _L)]
                addr = a * 256 + (lane16 + p)
                plsc.addupdate_scatter(hist_v, [addr], ones)

        for a in range(2):
            acc = hist_v[pl.ds(a * 256, _L)]
            for l in range(1, _L):
                acc = acc + hist_v[pl.ds(a * 256 + l * _L, _L)]
            row_v[...] = acc
            pltpu.sync_copy(row_v,
                            out_hbm.at[pl.ds((a * _NW + wid) * _L, _L)])

    return hist_kernel


def _spd_body(x_ref, o_ref):
    h0 = x_ref[pl.ds(0, _L)]
    h1 = x_ref[pl.ds(_NW * _L, _L)]
    for w in range(1, _NW):
        h0 = h0 + x_ref[pl.ds(w * _L, _L)]
        h1 = h1 + x_ref[pl.ds((_NW + w) * _L, _L)]
    n0 = jnp.sum(h0)
    n1 = jnp.sum(h1)
    d = h0 / n0 - h1 / n1
    o_ref[0, 0] = jnp.sum(d * d)


@jax.jit
def kernel(preds, attrs):
    n = preds.shape[0]
    partial = _make_hist_kernel(n)(preds, attrs)
    spd = pl.pallas_call(
        _spd_body,
        out_shape=jax.ShapeDtypeStruct((1, 1), jnp.float32),
        out_specs=pl.BlockSpec(memory_space=pltpu.SMEM),
    )(partial)
    return spd[0, 0]
